# baseline (speedup 1.0000x reference)
import functools

import jax
import jax.numpy as jnp
from jax import lax
from jax.experimental import pallas as pl
from jax.experimental.pallas import tpu as pltpu
from jax.experimental.pallas import tpu_sc as plsc

N = 10000
D = 128
E = 320000
HID = 256

NC = 2
NS = 16
NW = NC * NS
EDGES_PER_TILE = E // NW
CH = 80
NCH = EDGES_PER_TILE // CH
PAD_N = 10240
ROWS_PER_TILE = PAD_N // NS
ZROWS = 128


def _sc_aggregate(h, src, dst, zeros):
    mesh = plsc.VectorSubcoreMesh(core_axis_name="c", subcore_axis_name="s")

    @functools.partial(
        pl.kernel,
        mesh=mesh,
        out_type=jax.ShapeDtypeStruct((NC, PAD_N, D), jnp.float32),
        scratch_types=(
            [pltpu.VMEM((CH,), jnp.int32) for _ in range(3)]
            + [pltpu.VMEM((CH,), jnp.int32) for _ in range(3)]
            + [pltpu.VMEM((CH, D), jnp.float32) for _ in range(3)]
            + [pltpu.VMEM_SHARED((PAD_N, D), jnp.float32)]
            + [pltpu.SemaphoreType.DMA for _ in range(9)]
        ),
    )
    def agg_kernel(h_hbm, src_hbm, dst_hbm, z_hbm, out_hbm,
                   src0, src1, src2, dst0, dst1, dst2, row0_v, row1_v, row2_v,
                   acc_sh,
                   is0, is1, is2, gs0, gs1, gs2, ss0, ss1, ss2):
        cid = lax.axis_index("c")
        sid = lax.axis_index("s")
        base = (cid * NS + sid) * EDGES_PER_TILE
        r0 = sid * ROWS_PER_TILE

        srcs = [src0, src1, src2]
        dsts = [dst0, dst1, dst2]
        rows = [row0_v, row1_v, row2_v]
        isem = [is0, is1, is2]
        gsem = [gs0, gs1, gs2]
        ssem = [ss0, ss1, ss2]

        def idx_start(j, x):
            pltpu.async_copy(src_hbm.at[pl.ds(base + j * CH, CH)], srcs[x], isem[x])
            pltpu.async_copy(dst_hbm.at[pl.ds(base + j * CH, CH)], dsts[x], isem[x])

        def idx_wait(j, x):
            pltpu.make_async_copy(src_hbm.at[pl.ds(base + j * CH, CH)], srcs[x], isem[x]).wait()
            pltpu.make_async_copy(dst_hbm.at[pl.ds(base + j * CH, CH)], dsts[x], isem[x]).wait()

        def gather_start(x):
            pltpu.async_copy(h_hbm.at[srcs[x]], rows[x], gsem[x])

        def gather_wait(x):
            pltpu.make_async_copy(h_hbm.at[srcs[x]], rows[x], gsem[x]).wait()

        def scatter_start(x):
            pltpu.async_copy(rows[x], acc_sh.at[dsts[x]], ssem[x], add=True)

        def scatter_wait(x):
            pltpu.make_async_copy(rows[x], acc_sh.at[dsts[x]], ssem[x]).wait()

        def emit(c, s, do_c=True, do_d=True, do_e=True):
            sp = (s + 2) % 3
            sn = (s + 1) % 3
            gather_wait(s)
            scatter_start(s)
            if do_c:
                scatter_wait(sp)
            if do_d:
                idx_start(c + 2, sp)
            if do_e:
                idx_wait(c + 1, sn)
                gather_start(sn)

        idx_start(0, 0)
        idx_start(1, 1)
        idx_wait(0, 0)
        gather_start(0)
        idx_start(2, 2)
        pltpu.sync_copy(z_hbm.at[pl.ds(r0, ROWS_PER_TILE)],
                        acc_sh.at[pl.ds(r0, ROWS_PER_TILE)])
        plsc.subcore_barrier()

        emit(0, 0, do_c=False, do_d=False)
        emit(1, 1)

        @pl.loop(0, (NCH - 5) // 3)
        def _(i):
            c0 = 2 + 3 * i
            emit(c0, 2)
            emit(c0 + 1, 0)
            emit(c0 + 2, 1)

        emit(NCH - 3, 2)
        emit(NCH - 2, 0, do_d=False)
        emit(NCH - 1, 1, do_d=False, do_e=False)
        scatter_wait(1)

        plsc.subcore_barrier()

        pltpu.sync_copy(acc_sh.at[pl.ds(r0, ROWS_PER_TILE)],
                        out_hbm.at[cid, pl.ds(r0, ROWS_PER_TILE)])

    return agg_kernel(h, src, dst, zeros)


def _tc_mlp(h, agg0, agg1, W1, b1, W2, b2, act):
    BN = 1000

    def mlp_kernel(h_ref, a0_ref, a1_ref, W1_ref, b1_ref, W2_ref, b2_ref, o_ref):
        z = h_ref[...] + a0_ref[...] + a1_ref[...]
        t = jnp.dot(z, W1_ref[...], preferred_element_type=jnp.float32)
        t = t + b1_ref[...]
        t = jnp.where(t > 0, t, 0.2 * t)
        o = jnp.dot(t, W2_ref[...], preferred_element_type=jnp.float32)
        o = o + b2_ref[...]
        if act:
            o = jnp.where(o > 0, o, 0.2 * o)
        o_ref[...] = o

    return pl.pallas_call(
        mlp_kernel,
        grid=(N // BN,),
        in_specs=[
            pl.BlockSpec((BN, D), lambda i: (i, 0)),
            pl.BlockSpec((BN, D), lambda i: (i, 0)),
            pl.BlockSpec((BN, D), lambda i: (i, 0)),
            pl.BlockSpec((D, HID), lambda i: (0, 0)),
            pl.BlockSpec((1, HID), lambda i: (0, 0)),
            pl.BlockSpec((HID, D), lambda i: (0, 0)),
            pl.BlockSpec((1, D), lambda i: (0, 0)),
        ],
        out_specs=pl.BlockSpec((BN, D), lambda i: (i, 0)),
        out_shape=jax.ShapeDtypeStruct((N, D), jnp.float32),
    )(h, agg0, agg1, W1, b1.reshape(1, HID), W2, b2.reshape(1, D))


def kernel(x, edge_index,
           W1_0, b1_0, W2_0, b2_0,
           W1_1, b1_1, W2_1, b2_1,
           W1_2, b1_2, W2_2, b2_2):
    src = edge_index[0]
    dst = edge_index[1]
    zeros = jnp.zeros((PAD_N, D), jnp.float32)
    params = [(W1_0, b1_0, W2_0, b2_0),
              (W1_1, b1_1, W2_1, b2_1),
              (W1_2, b1_2, W2_2, b2_2)]
    h = x
    for l in range(3):
        agg = _sc_aggregate(h, src, dst, zeros)
        h = _tc_mlp(h, agg[0], agg[1], *params[l], act=(l < 2))
    return h


# speedup vs baseline: 1.0033x; 1.0033x over previous
import functools

import jax
import jax.numpy as jnp
from jax import lax
from jax.experimental import pallas as pl
from jax.experimental.pallas import tpu as pltpu
from jax.experimental.pallas import tpu_sc as plsc

N = 10000
D = 128
E = 320000
HID = 256

NC = 2
NS = 16
NW = NC * NS
EDGES_PER_TILE = E // NW
CH = 80
NCH = EDGES_PER_TILE // CH
PAD_N = 10240
ROWS_PER_TILE = PAD_N // NS
ZROWS = 128


def _sc_aggregate(h, src, dst, zeros):
    mesh = plsc.VectorSubcoreMesh(core_axis_name="c", subcore_axis_name="s")

    @functools.partial(
        pl.kernel,
        mesh=mesh,
        out_type=jax.ShapeDtypeStruct((NC, PAD_N, D), jnp.float32),
        scratch_types=(
            [pltpu.VMEM((CH,), jnp.int32) for _ in range(3)]
            + [pltpu.VMEM((CH,), jnp.int32) for _ in range(3)]
            + [pltpu.VMEM((CH, D), jnp.float32) for _ in range(3)]
            + [pltpu.VMEM_SHARED((PAD_N, D), jnp.float32)]
            + [pltpu.SemaphoreType.DMA for _ in range(9)]
        ),
    )
    def agg_kernel(h_hbm, src_hbm, dst_hbm, z_hbm, out_hbm,
                   src0, src1, src2, dst0, dst1, dst2, row0_v, row1_v, row2_v,
                   acc_sh,
                   is0, is1, is2, gs0, gs1, gs2, ss0, ss1, ss2):
        cid = lax.axis_index("c")
        sid = lax.axis_index("s")
        base = (cid * NS + sid) * EDGES_PER_TILE
        r0 = sid * ROWS_PER_TILE

        srcs = [src0, src1, src2]
        dsts = [dst0, dst1, dst2]
        rows = [row0_v, row1_v, row2_v]
        isem = [is0, is1, is2]
        gsem = [gs0, gs1, gs2]
        ssem = [ss0, ss1, ss2]

        def idx_start(j, x):
            pltpu.async_copy(src_hbm.at[pl.ds(base + j * CH, CH)], srcs[x], isem[x])
            pltpu.async_copy(dst_hbm.at[pl.ds(base + j * CH, CH)], dsts[x], isem[x])

        def idx_wait(j, x):
            pltpu.make_async_copy(src_hbm.at[pl.ds(base + j * CH, CH)], srcs[x], isem[x]).wait()
            pltpu.make_async_copy(dst_hbm.at[pl.ds(base + j * CH, CH)], dsts[x], isem[x]).wait()

        def gather_start(x):
            pltpu.async_copy(h_hbm.at[srcs[x]], rows[x], gsem[x])

        def gather_wait(x):
            pltpu.make_async_copy(h_hbm.at[srcs[x]], rows[x], gsem[x]).wait()

        def scatter_start(x):
            pltpu.async_copy(rows[x], acc_sh.at[dsts[x]], ssem[x], add=True)

        def scatter_wait(x):
            pltpu.make_async_copy(rows[x], acc_sh.at[dsts[x]], ssem[x]).wait()

        def emit(c, s, do_c=True, do_d=True, do_e=True):
            sp = (s + 2) % 3
            sn = (s + 1) % 3
            gather_wait(s)
            if do_e:
                idx_wait(c + 1, sn)
                gather_start(sn)
            scatter_start(s)
            if do_c:
                scatter_wait(sp)
            if do_d:
                idx_start(c + 2, sp)

        idx_start(0, 0)
        idx_start(1, 1)
        idx_wait(0, 0)
        gather_start(0)
        idx_start(2, 2)
        pltpu.sync_copy(z_hbm.at[pl.ds(r0, ROWS_PER_TILE)],
                        acc_sh.at[pl.ds(r0, ROWS_PER_TILE)])
        plsc.subcore_barrier()

        emit(0, 0, do_c=False, do_d=False)
        emit(1, 1)

        @pl.loop(0, (NCH - 5) // 3)
        def _(i):
            c0 = 2 + 3 * i
            emit(c0, 2)
            emit(c0 + 1, 0)
            emit(c0 + 2, 1)

        emit(NCH - 3, 2)
        emit(NCH - 2, 0, do_d=False)
        emit(NCH - 1, 1, do_d=False, do_e=False)
        scatter_wait(1)

        plsc.subcore_barrier()

        pltpu.sync_copy(acc_sh.at[pl.ds(r0, ROWS_PER_TILE)],
                        out_hbm.at[cid, pl.ds(r0, ROWS_PER_TILE)])

    return agg_kernel(h, src, dst, zeros)


def _tc_mlp(h, agg0, agg1, W1, b1, W2, b2, act):
    BN = 1000

    def mlp_kernel(h_ref, a0_ref, a1_ref, W1_ref, b1_ref, W2_ref, b2_ref, o_ref):
        z = h_ref[...] + a0_ref[...] + a1_ref[...]
        t = jnp.dot(z, W1_ref[...], preferred_element_type=jnp.float32)
        t = t + b1_ref[...]
        t = jnp.where(t > 0, t, 0.2 * t)
        o = jnp.dot(t, W2_ref[...], preferred_element_type=jnp.float32)
        o = o + b2_ref[...]
        if act:
            o = jnp.where(o > 0, o, 0.2 * o)
        o_ref[...] = o

    return pl.pallas_call(
        mlp_kernel,
        grid=(N // BN,),
        in_specs=[
            pl.BlockSpec((BN, D), lambda i: (i, 0)),
            pl.BlockSpec((BN, D), lambda i: (i, 0)),
            pl.BlockSpec((BN, D), lambda i: (i, 0)),
            pl.BlockSpec((D, HID), lambda i: (0, 0)),
            pl.BlockSpec((1, HID), lambda i: (0, 0)),
            pl.BlockSpec((HID, D), lambda i: (0, 0)),
            pl.BlockSpec((1, D), lambda i: (0, 0)),
        ],
        out_specs=pl.BlockSpec((BN, D), lambda i: (i, 0)),
        out_shape=jax.ShapeDtypeStruct((N, D), jnp.float32),
    )(h, agg0, agg1, W1, b1.reshape(1, HID), W2, b2.reshape(1, D))


def kernel(x, edge_index,
           W1_0, b1_0, W2_0, b2_0,
           W1_1, b1_1, W2_1, b2_1,
           W1_2, b1_2, W2_2, b2_2):
    src = edge_index[0]
    dst = edge_index[1]
    zeros = jnp.zeros((PAD_N, D), jnp.float32)
    params = [(W1_0, b1_0, W2_0, b2_0),
              (W1_1, b1_1, W2_1, b2_1),
              (W1_2, b1_2, W2_2, b2_2)]
    h = x
    for l in range(3):
        agg = _sc_aggregate(h, src, dst, zeros)
        h = _tc_mlp(h, agg[0], agg[1], *params[l], act=(l < 2))
    return h
